# baseline (speedup 1.0000x reference)
import functools

import jax
import jax.numpy as jnp
from jax import lax
from jax.experimental import pallas as pl
from jax.experimental.pallas import tpu as pltpu
from jax.experimental.pallas import tpu_sc as plsc

N = 10000
E = 320000
D = 128
T = 16
NC = 2
NS = 16
NW = NC * NS
NPAD = 10240
RPT = NPAD // NW
RROWS = NPAD // NS
EC = 128
EPT = 10240
EPAD = NW * EPT
NCHUNK = EPT // EC
NBUF = 2
GCHUNK = 80

def _mesh():
    return plsc.VectorSubcoreMesh(
        core_axis_name="c", subcore_axis_name="s", num_cores=NC, num_subcores=NS)


def _worker_id():
    return lax.axis_index("s") * NC + lax.axis_index("c")


@functools.cache
def _row_gather():
    @functools.partial(
        pl.kernel,
        out_type=jax.ShapeDtypeStruct((NPAD, D), jnp.float32),
        mesh=_mesh(),
        scratch_types=[
            pltpu.VMEM((GCHUNK,), jnp.int32),
            pltpu.VMEM((GCHUNK, D), jnp.float32),
            pltpu.SemaphoreType.DMA,
        ],
    )
    def k(table_hbm, idx_hbm, out_hbm, idx_v, rows_v, sem):
        base = _worker_id() * RPT
        for j in range(RPT // GCHUNK):
            off = base + j * GCHUNK
            pltpu.sync_copy(idx_hbm.at[pl.ds(off, GCHUNK)], idx_v)
            pltpu.async_copy(table_hbm.at[idx_v], rows_v, sem).wait()
            pltpu.sync_copy(rows_v, out_hbm.at[pl.ds(off, GCHUNK)])

    return k


@functools.cache
def _sc_aggregate():
    @functools.partial(
        pl.kernel,
        out_type=jax.ShapeDtypeStruct((NC, NPAD, D), jnp.float32),
        mesh=_mesh(),
        scratch_types=(
            [pltpu.VMEM((EC,), jnp.int32) for _ in range(NBUF)]
            + [pltpu.VMEM((EC,), jnp.int32) for _ in range(NBUF)]
            + [pltpu.VMEM((EC, D), jnp.float32) for _ in range(NBUF)]
            + [pltpu.VMEM_SHARED((NPAD, D), jnp.float32)]
            + [pltpu.SemaphoreType.DMA for _ in range(3 * NBUF)]
        ),
    )
    def k(h16_hbm, cidx_hbm, dst_hbm, zeros_hbm, out_hbm, *refs):
        cidx_v = refs[0:NBUF]
        didx_v = refs[NBUF:2 * NBUF]
        rows_v = refs[2 * NBUF:3 * NBUF]
        aggr_sh = refs[3 * NBUF]
        semi = refs[3 * NBUF + 1:3 * NBUF + 1 + NBUF]
        semg = refs[3 * NBUF + 1 + NBUF:3 * NBUF + 1 + 2 * NBUF]
        sems = refs[3 * NBUF + 1 + 2 * NBUF:3 * NBUF + 1 + 3 * NBUF]
        c = lax.axis_index("c")
        s = lax.axis_index("s")
        w = s * NC + c
        pltpu.sync_copy(zeros_hbm.at[pl.ds(s * RROWS, RROWS)],
                        aggr_sh.at[pl.ds(s * RROWS, RROWS)])
        plsc.subcore_barrier()
        ebase = w * EPT

        def scatter_desc(b):
            return pltpu.make_async_copy(rows_v[b], aggr_sh.at[didx_v[b]],
                                         sems[b])

        def ring(j, carry):
            base = ebase + j * (NBUF * EC)
            for b in range(NBUF):
                @pl.when(j > 0)
                def _drain(b=b):
                    scatter_desc(b).wait()

                off = base + b * EC
                pltpu.async_copy(cidx_hbm.at[pl.ds(off, EC)], cidx_v[b],
                                 semi[b])
                pltpu.async_copy(dst_hbm.at[pl.ds(off, EC)], didx_v[b],
                                 semi[b])
            for b in range(NBUF):
                pltpu.make_async_copy(cidx_hbm.at[pl.ds(0, EC)], cidx_v[b],
                                      semi[b]).wait()
                pltpu.make_async_copy(dst_hbm.at[pl.ds(0, EC)], didx_v[b],
                                      semi[b]).wait()
                pltpu.async_copy(h16_hbm.at[cidx_v[b]], rows_v[b], semg[b])
            for b in range(NBUF):
                pltpu.make_async_copy(h16_hbm.at[cidx_v[b]], rows_v[b],
                                      semg[b]).wait()
                scatter_desc(b).start(add=True)
            return carry

        lax.fori_loop(0, NCHUNK // NBUF, ring, 0)
        for b in range(NBUF):
            scatter_desc(b).wait()
        plsc.subcore_barrier()
        pltpu.sync_copy(aggr_sh.at[pl.ds(s * RROWS, RROWS)],
                        out_hbm.at[c, pl.ds(s * RROWS, RROWS)])

    return k


_NB = 25
_BR = N // _NB


def _build_body(h_ref, gate_ref, out_ref):
    out_ref[0] = h_ref[...] * gate_ref[0]


_tc_build = pl.pallas_call(
    _build_body,
    grid=(_NB, T),
    in_specs=[
        pl.BlockSpec((_BR, D), lambda i, t: (i, 0)),
        pl.BlockSpec((1, 1, D), lambda i, t: (t, 0, 0)),
    ],
    out_specs=pl.BlockSpec((1, _BR, D), lambda i, t: (t, i, 0)),
    out_shape=jax.ShapeDtypeStruct((T, N, D), jnp.float32),
)


def _gru_core(a_ref, dg_ref, nt_ref, h_ref, wih_ref, whh_ref, bih_ref,
              bhh_ref):
    a = a_ref[0] + a_ref[1]
    dg = dg_ref[0, :, 0:1] + dg_ref[1, :, 0:1]
    dg = jnp.maximum(dg, 1.0)
    inp = a / dg + nt_ref[...]
    h = h_ref[...]
    gi = jnp.dot(inp, wih_ref[...], preferred_element_type=jnp.float32)
    gi = gi + bih_ref[...]
    gh = jnp.dot(h, whh_ref[...], preferred_element_type=jnp.float32)
    gh = gh + bhh_ref[...]
    r = jax.nn.sigmoid(gi[:, 0:D] + gh[:, 0:D])
    z = jax.nn.sigmoid(gi[:, D:2 * D] + gh[:, D:2 * D])
    n = jnp.tanh(gi[:, 2 * D:] + r * gh[:, 2 * D:])
    return (1.0 - z) * n + z * h


def _gru_body(a_ref, dg_ref, nt_ref, h_ref, wih_ref, whh_ref, bih_ref,
              bhh_ref, out_ref):
    out_ref[...] = _gru_core(a_ref, dg_ref, nt_ref, h_ref, wih_ref, whh_ref,
                             bih_ref, bhh_ref)


def _gru_fused_body(a_ref, dg_ref, nt_ref, h_ref, wih_ref, whh_ref, bih_ref,
                    bhh_ref, gate_ref, out_ref, out16_ref):
    hn = _gru_core(a_ref, dg_ref, nt_ref, h_ref, wih_ref, whh_ref,
                   bih_ref, bhh_ref)
    out_ref[...] = hn
    for t in range(T):
        out16_ref[t] = hn * gate_ref[t]


_GRU_SPECS = [
    pl.BlockSpec((NC, _BR, D), lambda i: (0, i, 0)),
    pl.BlockSpec((NC, _BR, D), lambda i: (0, i, 0)),
    pl.BlockSpec((_BR, D), lambda i: (i, 0)),
    pl.BlockSpec((_BR, D), lambda i: (i, 0)),
    pl.BlockSpec((D, 3 * D), lambda i: (0, 0)),
    pl.BlockSpec((D, 3 * D), lambda i: (0, 0)),
    pl.BlockSpec((1, 3 * D), lambda i: (0, 0)),
    pl.BlockSpec((1, 3 * D), lambda i: (0, 0)),
]

_tc_gru = pl.pallas_call(
    _gru_body,
    grid=(_NB,),
    in_specs=_GRU_SPECS,
    out_specs=pl.BlockSpec((_BR, D), lambda i: (i, 0)),
    out_shape=jax.ShapeDtypeStruct((N, D), jnp.float32),
)

_tc_gru_fused = pl.pallas_call(
    _gru_fused_body,
    grid=(_NB,),
    in_specs=_GRU_SPECS + [pl.BlockSpec((T, 1, D), lambda i: (0, 0, 0))],
    out_specs=[
        pl.BlockSpec((_BR, D), lambda i: (i, 0)),
        pl.BlockSpec((T, _BR, D), lambda i: (0, i, 0)),
    ],
    out_shape=[
        jax.ShapeDtypeStruct((N, D), jnp.float32),
        jax.ShapeDtypeStruct((T, N, D), jnp.float32),
    ],
)


def kernel(x, edge_index, edge_type, node_type, emb, edge_type_emb,
           node_type_bias, W_ih, W_hh, b_ih, b_hh):
    src = edge_index[0].astype(jnp.int32)
    dst = edge_index[1].astype(jnp.int32)
    et = edge_type.astype(jnp.int32)
    cidx = et * N + src
    cidx_p = jnp.pad(cidx, (0, EPAD - E))
    dst_p = jnp.pad(dst, (0, EPAD - E), constant_values=N)
    x_p = jnp.pad(x.astype(jnp.int32), (0, NPAD - N))
    nt_p = jnp.pad(node_type.astype(jnp.int32), (0, NPAD - N))
    zeros128 = jnp.zeros((NPAD, D), jnp.float32)
    ones_table = jnp.ones((T * N, D), jnp.float32)
    wihT = W_ih.T
    whhT = W_hh.T
    bih2 = b_ih.reshape(1, 3 * D)
    bhh2 = b_hh.reshape(1, 3 * D)

    gather = _row_gather()
    h = gather(emb, x_p)[:N]
    nt_rows = gather(node_type_bias, nt_p)[:N]
    gate3 = edge_type_emb.reshape(T, 1, D)
    aggregate = _sc_aggregate()
    degp = aggregate(ones_table, cidx_p, dst_p, zeros128)
    h16 = _tc_build(h, gate3)
    for it in range(5):
        aggp = aggregate(h16.reshape(T * N, D), cidx_p, dst_p, zeros128)
        if it < 4:
            h, h16 = _tc_gru_fused(aggp, degp, nt_rows, h, wihT, whhT,
                                   bih2, bhh2, gate3)
        else:
            h = _tc_gru(aggp, degp, nt_rows, h, wihT, whhT, bih2, bhh2)
    return h


# speedup vs baseline: 1.0974x; 1.0974x over previous
import functools

import jax
import jax.numpy as jnp
from jax import lax
from jax.experimental import pallas as pl
from jax.experimental.pallas import tpu as pltpu
from jax.experimental.pallas import tpu_sc as plsc

N = 10000
E = 320000
D = 128
T = 16
NC = 2
NS = 16
NW = NC * NS
NPAD = 10240
RPT = NPAD // NW
RROWS = NPAD // NS
EC = 128
EPT = 10240
EPAD = NW * EPT
NCHUNK = EPT // EC
NRB = 2
NIB = 4
GCHUNK = 80

def _mesh():
    return plsc.VectorSubcoreMesh(
        core_axis_name="c", subcore_axis_name="s", num_cores=NC, num_subcores=NS)


def _worker_id():
    return lax.axis_index("s") * NC + lax.axis_index("c")


@functools.cache
def _row_gather():
    @functools.partial(
        pl.kernel,
        out_type=jax.ShapeDtypeStruct((NPAD, D), jnp.float32),
        mesh=_mesh(),
        scratch_types=[
            pltpu.VMEM((GCHUNK,), jnp.int32),
            pltpu.VMEM((GCHUNK, D), jnp.float32),
            pltpu.SemaphoreType.DMA,
        ],
    )
    def k(table_hbm, idx_hbm, out_hbm, idx_v, rows_v, sem):
        base = _worker_id() * RPT
        for j in range(RPT // GCHUNK):
            off = base + j * GCHUNK
            pltpu.sync_copy(idx_hbm.at[pl.ds(off, GCHUNK)], idx_v)
            pltpu.async_copy(table_hbm.at[idx_v], rows_v, sem).wait()
            pltpu.sync_copy(rows_v, out_hbm.at[pl.ds(off, GCHUNK)])

    return k


@functools.cache
def _sc_aggregate():
    @functools.partial(
        pl.kernel,
        out_type=jax.ShapeDtypeStruct((NC, NPAD, D), jnp.float32),
        mesh=_mesh(),
        scratch_types=(
            [pltpu.VMEM((EC,), jnp.int32) for _ in range(NIB)]
            + [pltpu.VMEM((EC,), jnp.int32) for _ in range(NIB)]
            + [pltpu.VMEM((EC, D), jnp.float32) for _ in range(NRB)]
            + [pltpu.VMEM_SHARED((NPAD, D), jnp.float32)]
            + [pltpu.SemaphoreType.DMA for _ in range(NIB + 2 * NRB)]
        ),
    )
    def k(h16_hbm, cidx_hbm, dst_hbm, zeros_hbm, out_hbm, *refs):
        cidx_v = refs[0:NIB]
        didx_v = refs[NIB:2 * NIB]
        rows_v = refs[2 * NIB:2 * NIB + NRB]
        aggr_sh = refs[2 * NIB + NRB]
        o = 2 * NIB + NRB + 1
        semi = refs[o:o + NIB]
        semg = refs[o + NIB:o + NIB + NRB]
        sems = refs[o + NIB + NRB:o + NIB + 2 * NRB]
        c = lax.axis_index("c")
        s = lax.axis_index("s")
        w = s * NC + c
        pltpu.sync_copy(zeros_hbm.at[pl.ds(s * RROWS, RROWS)],
                        aggr_sh.at[pl.ds(s * RROWS, RROWS)])
        plsc.subcore_barrier()
        ebase = w * EPT

        def issue_idx(k_off, q):
            pltpu.async_copy(cidx_hbm.at[pl.ds(k_off, EC)], cidx_v[q], semi[q])
            pltpu.async_copy(dst_hbm.at[pl.ds(k_off, EC)], didx_v[q], semi[q])

        def wait_idx(q):
            pltpu.make_async_copy(cidx_hbm.at[pl.ds(0, EC)], cidx_v[q],
                                  semi[q]).wait()
            pltpu.make_async_copy(dst_hbm.at[pl.ds(0, EC)], didx_v[q],
                                  semi[q]).wait()

        def issue_gather(q, r):
            pltpu.async_copy(h16_hbm.at[cidx_v[q]], rows_v[r], semg[r])

        def wait_gather(q, r):
            pltpu.make_async_copy(h16_hbm.at[cidx_v[q]], rows_v[r],
                                  semg[r]).wait()

        def scatter_desc(r, q):
            return pltpu.make_async_copy(rows_v[r], aggr_sh.at[didx_v[q]],
                                         sems[r])

        def chunk_body(k, koff, first):
            r, q = k % 2, k % 4
            if not first:
                scatter_desc(r, (k - 2) % 4).wait()
            wait_idx(q)
            issue_gather(q, r)
            if not first:
                pr, pq = (k - 1) % 2, (k - 1) % 4
                wait_gather(pq, pr)
                scatter_desc(pr, pq).start(add=True)
            if k + 2 < NCHUNK:
                issue_idx(koff + 2 * EC, (k + 2) % 4)

        issue_idx(ebase, 0)
        issue_idx(ebase + EC, 1)
        chunk_body(0, ebase, True)
        wait_idx(1)
        issue_gather(1, 1)
        issue_idx(ebase + 3 * EC, 3)
        wait_gather(0, 0)
        scatter_desc(0, 0).start(add=True)

        def ring(j, carry):
            base = ebase + j * (4 * EC) + 2 * EC
            for i in range(4):
                k = i + 2
                r, q = k % 2, k % 4
                scatter_desc(r, (k - 2) % 4).wait()
                wait_idx(q)
                issue_gather(q, r)
                pr, pq = (k - 1) % 2, (k - 1) % 4
                wait_gather(pq, pr)
                scatter_desc(pr, pq).start(add=True)
                koff = base + i * EC
                pltpu.async_copy(cidx_hbm.at[pl.ds(koff + 2 * EC, EC)],
                                 cidx_v[(k + 2) % 4], semi[(k + 2) % 4])
                pltpu.async_copy(dst_hbm.at[pl.ds(koff + 2 * EC, EC)],
                                 didx_v[(k + 2) % 4], semi[(k + 2) % 4])
            return carry

        lax.fori_loop(0, (NCHUNK - 4) // 4, ring, 0)

        for k in (NCHUNK - 2, NCHUNK - 1):
            r, q = k % 2, k % 4
            scatter_desc(r, (k - 2) % 4).wait()
            wait_idx(q)
            issue_gather(q, r)
            pr, pq = (k - 1) % 2, (k - 1) % 4
            wait_gather(pq, pr)
            scatter_desc(pr, pq).start(add=True)
        lk = NCHUNK - 1
        wait_gather(lk % 4, lk % 2)
        scatter_desc(lk % 2, lk % 4).start(add=True)
        scatter_desc((lk - 1) % 2, (lk - 1) % 4).wait()
        scatter_desc(lk % 2, lk % 4).wait()
        plsc.subcore_barrier()
        pltpu.sync_copy(aggr_sh.at[pl.ds(s * RROWS, RROWS)],
                        out_hbm.at[c, pl.ds(s * RROWS, RROWS)])

    return k


_NB = 25
_BR = N // _NB


def _build_body(h_ref, gate_ref, out_ref):
    out_ref[0] = h_ref[...] * gate_ref[0]


_tc_build = pl.pallas_call(
    _build_body,
    grid=(_NB, T),
    in_specs=[
        pl.BlockSpec((_BR, D), lambda i, t: (i, 0)),
        pl.BlockSpec((1, 1, D), lambda i, t: (t, 0, 0)),
    ],
    out_specs=pl.BlockSpec((1, _BR, D), lambda i, t: (t, i, 0)),
    out_shape=jax.ShapeDtypeStruct((T, N, D), jnp.float32),
)


def _gru_core(a_ref, dg_ref, nt_ref, h_ref, wih_ref, whh_ref, bih_ref,
              bhh_ref):
    a = a_ref[0] + a_ref[1]
    dg = dg_ref[0, :, 0:1] + dg_ref[1, :, 0:1]
    dg = jnp.maximum(dg, 1.0)
    inp = a / dg + nt_ref[...]
    h = h_ref[...]
    gi = jnp.dot(inp, wih_ref[...], preferred_element_type=jnp.float32)
    gi = gi + bih_ref[...]
    gh = jnp.dot(h, whh_ref[...], preferred_element_type=jnp.float32)
    gh = gh + bhh_ref[...]
    r = jax.nn.sigmoid(gi[:, 0:D] + gh[:, 0:D])
    z = jax.nn.sigmoid(gi[:, D:2 * D] + gh[:, D:2 * D])
    n = jnp.tanh(gi[:, 2 * D:] + r * gh[:, 2 * D:])
    return (1.0 - z) * n + z * h


def _gru_body(a_ref, dg_ref, nt_ref, h_ref, wih_ref, whh_ref, bih_ref,
              bhh_ref, out_ref):
    out_ref[...] = _gru_core(a_ref, dg_ref, nt_ref, h_ref, wih_ref, whh_ref,
                             bih_ref, bhh_ref)


def _gru_fused_body(a_ref, dg_ref, nt_ref, h_ref, wih_ref, whh_ref, bih_ref,
                    bhh_ref, gate_ref, out_ref, out16_ref):
    hn = _gru_core(a_ref, dg_ref, nt_ref, h_ref, wih_ref, whh_ref,
                   bih_ref, bhh_ref)
    out_ref[...] = hn
    for t in range(T):
        out16_ref[t] = hn * gate_ref[t]


_GRU_SPECS = [
    pl.BlockSpec((NC, _BR, D), lambda i: (0, i, 0)),
    pl.BlockSpec((NC, _BR, D), lambda i: (0, i, 0)),
    pl.BlockSpec((_BR, D), lambda i: (i, 0)),
    pl.BlockSpec((_BR, D), lambda i: (i, 0)),
    pl.BlockSpec((D, 3 * D), lambda i: (0, 0)),
    pl.BlockSpec((D, 3 * D), lambda i: (0, 0)),
    pl.BlockSpec((1, 3 * D), lambda i: (0, 0)),
    pl.BlockSpec((1, 3 * D), lambda i: (0, 0)),
]

_tc_gru = pl.pallas_call(
    _gru_body,
    grid=(_NB,),
    in_specs=_GRU_SPECS,
    out_specs=pl.BlockSpec((_BR, D), lambda i: (i, 0)),
    out_shape=jax.ShapeDtypeStruct((N, D), jnp.float32),
)

_tc_gru_fused = pl.pallas_call(
    _gru_fused_body,
    grid=(_NB,),
    in_specs=_GRU_SPECS + [pl.BlockSpec((T, 1, D), lambda i: (0, 0, 0))],
    out_specs=[
        pl.BlockSpec((_BR, D), lambda i: (i, 0)),
        pl.BlockSpec((T, _BR, D), lambda i: (0, i, 0)),
    ],
    out_shape=[
        jax.ShapeDtypeStruct((N, D), jnp.float32),
        jax.ShapeDtypeStruct((T, N, D), jnp.float32),
    ],
)


def kernel(x, edge_index, edge_type, node_type, emb, edge_type_emb,
           node_type_bias, W_ih, W_hh, b_ih, b_hh):
    src = edge_index[0].astype(jnp.int32)
    dst = edge_index[1].astype(jnp.int32)
    et = edge_type.astype(jnp.int32)
    cidx = et * N + src
    cidx_p = jnp.pad(cidx, (0, EPAD - E))
    dst_p = jnp.pad(dst, (0, EPAD - E), constant_values=N)
    x_p = jnp.pad(x.astype(jnp.int32), (0, NPAD - N))
    nt_p = jnp.pad(node_type.astype(jnp.int32), (0, NPAD - N))
    zeros128 = jnp.zeros((NPAD, D), jnp.float32)
    ones_table = jnp.ones((T * N, D), jnp.float32)
    wihT = W_ih.T
    whhT = W_hh.T
    bih2 = b_ih.reshape(1, 3 * D)
    bhh2 = b_hh.reshape(1, 3 * D)

    gather = _row_gather()
    h = gather(emb, x_p)[:N]
    nt_rows = gather(node_type_bias, nt_p)[:N]
    gate3 = edge_type_emb.reshape(T, 1, D)
    aggregate = _sc_aggregate()
    degp = aggregate(ones_table, cidx_p, dst_p, zeros128)
    h16 = _tc_build(h, gate3)
    for it in range(5):
        aggp = aggregate(h16.reshape(T * N, D), cidx_p, dst_p, zeros128)
        if it < 4:
            h, h16 = _tc_gru_fused(aggp, degp, nt_rows, h, wihT, whhT,
                                   bih2, bhh2, gate3)
        else:
            h = _tc_gru(aggp, degp, nt_rows, h, wihT, whhT, bih2, bhh2)
    return h
